# baseline (speedup 1.0000x reference)
import functools
import math

import jax
import jax.numpy as jnp
from jax import lax
from jax.experimental import pallas as pl
from jax.experimental.pallas import tpu as pltpu
from jax.experimental.pallas import tpu_sc as plsc

_LANES = 16


def _build_sc_embed(B, S, V, D, MAXS, NC, NS):
    NW = NC * NS
    pos_per_w = S // NW
    P = 8
    n_chunks = pos_per_w // P
    vecs_per_row = D // _LANES
    scale = math.sqrt(float(D))
    mesh = plsc.VectorSubcoreMesh(core_axis_name="c", subcore_axis_name="s")

    NBUF = 3

    @functools.partial(
        pl.kernel,
        out_type=jax.ShapeDtypeStruct((B, S, D), jnp.float32),
        mesh=mesh,
        scratch_types=[
            pltpu.VMEM((B, pos_per_w), jnp.int32),
            pltpu.VMEM((NBUF, B * P, D), jnp.float32),
            pltpu.VMEM((NBUF, P, D), jnp.float32),
        ] + [pltpu.SemaphoreType.DMA] * (2 * NBUF),
    )
    def sc_embed(x_hbm, table_hbm, pe_hbm, out_hbm, idx_v, rows_v, pe_v,
                 *sems):
        wid = lax.axis_index("s") * NC + lax.axis_index("c")
        pos_base = wid * pos_per_w
        gsems = sems[:NBUF]
        ssems = sems[NBUF:]

        pltpu.sync_copy(x_hbm.at[:, pl.ds(pos_base, pos_per_w)], idx_v)

        def gather_descs(c, buf):
            descs = []
            for b in range(B):
                descs.append((
                    table_hbm.at[idx_v.at[b, pl.ds(c * P, P)]],
                    rows_v.at[buf, pl.ds(b * P, P)],
                    gsems[buf]))
            descs.append((
                pe_hbm.at[0, pl.ds(pos_base + c * P, P)],
                pe_v.at[buf],
                gsems[buf]))
            return descs

        def store_descs(c, buf):
            return [(rows_v.at[buf, pl.ds(b * P, P)],
                     out_hbm.at[b, pl.ds(pos_base + c * P, P)],
                     ssems[buf])
                    for b in range(B)]

        for c0 in range(NBUF - 1):
            for src, dst, sem in gather_descs(c0, c0):
                pltpu.async_copy(src, dst, sem)

        n_t = (n_chunks - 1) // NBUF

        def compute_chunk(k):
            def vec_body(i, k=k):
                p = i // vecs_per_row
                j = i - p * vecs_per_row
                sl = pl.ds(j * _LANES, _LANES)
                pv = pe_v[k, p, sl]
                for b in range(B):
                    r = b * P + p
                    rows_v[k, r, sl] = rows_v[k, r, sl] * scale + pv

            plsc.parallel_loop(0, P * vecs_per_row, unroll=2)(vec_body)

        def round_body(t, carry):
            for k in range(NBUF):
                c = t * NBUF + k
                kprev = (k + NBUF - 1) % NBUF

                for src, dst, sem in gather_descs(c, k):
                    pltpu.make_async_copy(src, dst, sem).wait()
                compute_chunk(k)

                def _wait_prev_store(c=c, kprev=kprev):
                    for src, dst, sem in store_descs(c - 1, kprev):
                        pltpu.make_async_copy(src, dst, sem).wait()

                def _issue_next(c=c, kprev=kprev):
                    for src, dst, sem in gather_descs(c + NBUF - 1, kprev):
                        pltpu.async_copy(src, dst, sem)

                if k == 0:
                    pl.when(t >= 1)(_wait_prev_store)
                else:
                    _wait_prev_store()
                t_lim = -(-(n_chunks - k - NBUF + 1) // NBUF)
                if t_lim >= n_t:
                    _issue_next()
                else:
                    pl.when(t < t_lim)(_issue_next)

                for src, dst, sem in store_descs(c, k):
                    pltpu.async_copy(src, dst, sem)
            return carry

        lax.fori_loop(0, n_t, round_body, 0)
        for c in range(NBUF * n_t, n_chunks):
            k = c % NBUF
            for src, dst, sem in gather_descs(c, k):
                pltpu.make_async_copy(src, dst, sem).wait()
            compute_chunk(k)
            for src, dst, sem in store_descs(c, k):
                pltpu.async_copy(src, dst, sem)
        for c in range(NBUF * n_t - 1, n_chunks):
            for src, dst, sem in store_descs(c, c % NBUF):
                pltpu.make_async_copy(src, dst, sem).wait()

    return sc_embed


@jax.jit
def kernel(x, table, pe):
    B, S = x.shape
    V, D = table.shape
    info = plsc.get_sparse_core_info()
    sc_embed = _build_sc_embed(B, S, V, D, pe.shape[1],
                               info.num_cores, info.num_subcores)
    x32 = x.astype(jnp.int32)
    return sc_embed(x32, table, pe)


# speedup vs baseline: 1.0091x; 1.0091x over previous
import functools
import math

import jax
import jax.numpy as jnp
from jax import lax
from jax.experimental import pallas as pl
from jax.experimental.pallas import tpu as pltpu
from jax.experimental.pallas import tpu_sc as plsc

_LANES = 16


def _build_sc_embed(B, S, V, D, MAXS, NC, NS):
    NW = NC * NS
    pos_per_w = S // NW
    P = 8
    n_chunks = pos_per_w // P
    vecs_per_row = D // _LANES
    scale = math.sqrt(float(D))
    mesh = plsc.VectorSubcoreMesh(core_axis_name="c", subcore_axis_name="s")

    NBUF = 3

    @functools.partial(
        pl.kernel,
        out_type=jax.ShapeDtypeStruct((B, S, D), jnp.float32),
        mesh=mesh,
        scratch_types=[
            pltpu.VMEM((B, pos_per_w), jnp.int32),
            pltpu.VMEM((NBUF, B * P, D), jnp.float32),
            pltpu.VMEM((NBUF, P, D), jnp.float32),
        ] + [pltpu.SemaphoreType.DMA] * (2 * NBUF),
    )
    def sc_embed(x_hbm, table_hbm, pe_hbm, out_hbm, idx_v, rows_v, pe_v,
                 *sems):
        wid = lax.axis_index("s") * NC + lax.axis_index("c")
        pos_base = wid * pos_per_w
        gsems = sems[:NBUF]
        ssems = sems[NBUF:]

        pltpu.sync_copy(x_hbm.at[:, pl.ds(pos_base, pos_per_w)], idx_v)

        def gather_descs(c, buf):
            descs = []
            for b in range(B):
                descs.append((
                    table_hbm.at[idx_v.at[b, pl.ds(c * P, P)]],
                    rows_v.at[buf, pl.ds(b * P, P)],
                    gsems[buf]))
            descs.append((
                pe_hbm.at[0, pl.ds(pos_base + c * P, P)],
                pe_v.at[buf],
                gsems[buf]))
            return descs

        def store_descs(c, buf):
            return [(rows_v.at[buf, pl.ds(b * P, P)],
                     out_hbm.at[b, pl.ds(pos_base + c * P, P)],
                     ssems[buf])
                    for b in range(B)]

        for c0 in range(NBUF - 1):
            for src, dst, sem in gather_descs(c0, c0):
                pltpu.async_copy(src, dst, sem)

        n_t = (n_chunks - 1) // NBUF

        def compute_chunk(k):
            def vec_body(i, k=k):
                p = i // vecs_per_row
                j = i - p * vecs_per_row
                sl = pl.ds(j * _LANES, _LANES)
                pv = pe_v[k, p, sl]
                for b in range(B):
                    r = b * P + p
                    rows_v[k, r, sl] = rows_v[k, r, sl] * scale + pv

            plsc.parallel_loop(0, P * vecs_per_row, unroll=4)(vec_body)

        def round_body(t, carry):
            for k in range(NBUF):
                c = t * NBUF + k
                kprev = (k + NBUF - 1) % NBUF

                for src, dst, sem in gather_descs(c, k):
                    pltpu.make_async_copy(src, dst, sem).wait()
                compute_chunk(k)

                def _wait_prev_store(c=c, kprev=kprev):
                    for src, dst, sem in store_descs(c - 1, kprev):
                        pltpu.make_async_copy(src, dst, sem).wait()

                def _issue_next(c=c, kprev=kprev):
                    for src, dst, sem in gather_descs(c + NBUF - 1, kprev):
                        pltpu.async_copy(src, dst, sem)

                if k == 0:
                    pl.when(t >= 1)(_wait_prev_store)
                else:
                    _wait_prev_store()
                t_lim = -(-(n_chunks - k - NBUF + 1) // NBUF)
                if t_lim >= n_t:
                    _issue_next()
                else:
                    pl.when(t < t_lim)(_issue_next)

                for src, dst, sem in store_descs(c, k):
                    pltpu.async_copy(src, dst, sem)
            return carry

        lax.fori_loop(0, n_t, round_body, 0)
        for c in range(NBUF * n_t, n_chunks):
            k = c % NBUF
            for src, dst, sem in gather_descs(c, k):
                pltpu.make_async_copy(src, dst, sem).wait()
            compute_chunk(k)
            for src, dst, sem in store_descs(c, k):
                pltpu.async_copy(src, dst, sem)
        for c in range(NBUF * n_t - 1, n_chunks):
            for src, dst, sem in store_descs(c, c % NBUF):
                pltpu.make_async_copy(src, dst, sem).wait()

    return sc_embed


@jax.jit
def kernel(x, table, pe):
    B, S = x.shape
    V, D = table.shape
    info = plsc.get_sparse_core_info()
    sc_embed = _build_sc_embed(B, S, V, D, pe.shape[1],
                               info.num_cores, info.num_subcores)
    x32 = x.astype(jnp.int32)
    return sc_embed(x32, table, pe)
